# Initial kernel scaffold; baseline (speedup 1.0000x reference)
import jax
import jax.numpy as jnp
from jax.experimental import pallas as pl


def kernel(embeddings, positive_pairs, negative_pairs):
    raise NotImplementedError("write your pallas kernel here")


# speedup vs baseline: 9.2741x; 9.2741x over previous
import functools

import jax
import jax.numpy as jnp
from jax import lax
from jax.experimental import pallas as pl
from jax.experimental.pallas import tpu as pltpu
from jax.experimental.pallas import tpu_sc as plsc

_N_NODES = 10000
_D = 128
_N_PAIRS = 320000
_TOTAL_PAIRS = 2 * _N_PAIRS
_NC = 2
_NS = 16
_NW = _NC * _NS
_PAIRS_PER_W = _TOTAL_PAIRS // _NW
_C = 80
_NCHUNK = _PAIRS_PER_W // _C
_LANES = 16


def _norm_body(x_ref, o_ref):
    x = x_ref[...]
    n = jnp.sqrt(jnp.sum(x * x, axis=1, keepdims=True))
    o_ref[...] = x / jnp.maximum(n, 1e-8)


def _normalize(emb):
    blk = _N_NODES // 10
    return pl.pallas_call(
        _norm_body,
        out_shape=jax.ShapeDtypeStruct(emb.shape, emb.dtype),
        grid=(10,),
        in_specs=[pl.BlockSpec((blk, _D), lambda i: (i, 0))],
        out_specs=pl.BlockSpec((blk, _D), lambda i: (i, 0)),
    )(emb)


def _sc_body(emb_hbm, ia_hbm, ib_hbm, out_hbm,
             slab_a, slab_b, ra0, rb0, ra1, rb1, stage,
             sa0, sb0, sa1, sb1):
    wid = lax.axis_index("s") * _NC + lax.axis_index("c")

    pltpu.sync_copy(ia_hbm.at[wid], slab_a)
    pltpu.sync_copy(ib_hbm.at[wid], slab_b)

    def start_gather(c, ra, rb, sema, semb):
        pltpu.make_async_copy(emb_hbm.at[slab_a.at[c]], ra, sema).start()
        pltpu.make_async_copy(emb_hbm.at[slab_b.at[c]], rb, semb).start()

    def wait_gather(ra, rb, sema, semb):
        pltpu.make_async_copy(emb_hbm.at[pl.ds(0, _C)], ra, sema).wait()
        pltpu.make_async_copy(emb_hbm.at[pl.ds(0, _C)], rb, semb).wait()

    iota = lax.iota(jnp.int32, _LANES)
    pos_flag = (wid < (_NW // 2)).astype(jnp.float32)
    floor_vec = jnp.broadcast_to(-4.0 * pos_flag, (_LANES,))
    _UNROLL = 4

    def chunk_accum(ra, rb, acc):
        def pstep(t, pacc):
            for u in range(_UNROLL):
                p = t * _UNROLL + u
                v = ra[p, pl.ds(0, _LANES)] * rb[p, pl.ds(0, _LANES)]
                for k in range(1, _D // _LANES):
                    v = v + (ra[p, pl.ds(k * _LANES, _LANES)]
                             * rb[p, pl.ds(k * _LANES, _LANES)])
                for sh in (8, 4, 2, 1):
                    v = v + jnp.take(v, iota ^ sh, axis=0)
                pacc = pacc + jnp.maximum(v, floor_vec)
            return pacc

        return lax.fori_loop(0, _C // _UNROLL, pstep, acc)

    start_gather(0, ra0, rb0, sa0, sb0)
    start_gather(1, ra1, rb1, sa1, sb1)

    def body(k, acc):
        c0 = 2 * k
        wait_gather(ra0, rb0, sa0, sb0)
        acc = chunk_accum(ra0, rb0, acc)

        @pl.when(c0 + 2 < _NCHUNK)
        def _():
            start_gather(c0 + 2, ra0, rb0, sa0, sb0)

        wait_gather(ra1, rb1, sa1, sb1)
        acc = chunk_accum(ra1, rb1, acc)

        @pl.when(c0 + 3 < _NCHUNK)
        def _():
            start_gather(c0 + 3, ra1, rb1, sa1, sb1)

        return acc

    acc = lax.fori_loop(0, _NCHUNK // 2, body,
                        jnp.zeros((_LANES,), jnp.float32))

    stage[0, :] = acc
    pltpu.sync_copy(stage, out_hbm.at[wid])


@functools.partial(
    pl.kernel,
    mesh=plsc.VectorSubcoreMesh(core_axis_name="c", subcore_axis_name="s"),
    out_type=jax.ShapeDtypeStruct((_NW, 1, _LANES), jnp.float32),
    scratch_types=[
        pltpu.VMEM((_NCHUNK, _C), jnp.int32),
        pltpu.VMEM((_NCHUNK, _C), jnp.int32),
        pltpu.VMEM((_C, _D), jnp.float32),
        pltpu.VMEM((_C, _D), jnp.float32),
        pltpu.VMEM((_C, _D), jnp.float32),
        pltpu.VMEM((_C, _D), jnp.float32),
        pltpu.VMEM((1, _LANES), jnp.float32),
        pltpu.SemaphoreType.DMA,
        pltpu.SemaphoreType.DMA,
        pltpu.SemaphoreType.DMA,
        pltpu.SemaphoreType.DMA,
    ],
)
def _sc_loss(emb_hbm, ia_hbm, ib_hbm, out_hbm, *scratch):
    _sc_body(emb_hbm, ia_hbm, ib_hbm, out_hbm, *scratch)


def kernel(embeddings, positive_pairs, negative_pairs):
    emb_n = _normalize(embeddings)
    ia = jnp.concatenate(
        [positive_pairs[:, 0], negative_pairs[:, 0]]).reshape(_NW, _NCHUNK, _C)
    ib = jnp.concatenate(
        [positive_pairs[:, 1], negative_pairs[:, 1]]).reshape(_NW, _NCHUNK, _C)
    parts = _sc_loss(emb_n, ia, ib)
    pos_dot = jnp.sum(parts[: _NW // 2]) / _LANES
    neg_relu = jnp.sum(parts[_NW // 2:]) / _LANES
    return ((_N_PAIRS - pos_dot) + neg_relu) / _TOTAL_PAIRS
